# baseline (speedup 1.0000x reference)
import functools

import jax
import jax.numpy as jnp
from jax import lax
from jax.experimental import pallas as pl
from jax.experimental.pallas import tpu as pltpu
from jax.experimental.pallas import tpu_sc as plsc


def kernel(x, table):
    row = x.shape[1]
    emb = table.shape[1]

    mesh = plsc.VectorSubcoreMesh(core_axis_name="c", subcore_axis_name="s")

    @functools.partial(
        pl.kernel,
        mesh=mesh,
        out_type=jax.ShapeDtypeStruct((1, emb), table.dtype),
        scratch_types=[pltpu.VMEM((1, emb), table.dtype)],
    )
    def lookup(table_hbm, out_hbm, row_v):
        cid = lax.axis_index("c")
        sid = lax.axis_index("s")

        @pl.when(jnp.logical_and(cid == 0, sid == 0))
        def _():
            pltpu.sync_copy(table_hbm.at[pl.ds(row, 1), :], row_v)
            pltpu.sync_copy(row_v, out_hbm)

    return lookup(table).reshape((emb,))


# speedup vs baseline: 1.0207x; 1.0207x over previous
import functools

import jax
import jax.numpy as jnp
from jax import lax
from jax.experimental import pallas as pl
from jax.experimental.pallas import tpu as pltpu
from jax.experimental.pallas import tpu_sc as plsc


def kernel(x, table):
    row = x.shape[1]
    emb = table.shape[1]

    mesh = plsc.ScalarSubcoreMesh(axis_name="c", num_cores=2)

    @functools.partial(
        pl.kernel,
        mesh=mesh,
        out_type=jax.ShapeDtypeStruct((1, emb), table.dtype),
    )
    def lookup(table_hbm, out_hbm):
        cid = lax.axis_index("c")

        @pl.when(cid == 0)
        def _():
            pltpu.sync_copy(table_hbm.at[pl.ds(row, 1), :], out_hbm)

    return lookup(table).reshape((emb,))


# speedup vs baseline: 1.9082x; 1.8695x over previous
import jax
import jax.numpy as jnp
from jax.experimental import pallas as pl


def kernel(x, table):
    row = x.shape[1]
    emb = table.shape[1]

    def body(table_ref, out_ref):
        out_ref[0, :] = table_ref[row % 8, :]

    out = pl.pallas_call(
        body,
        out_shape=jax.ShapeDtypeStruct((1, emb), table.dtype),
        grid=(1,),
        in_specs=[pl.BlockSpec((8, emb), lambda i: (row // 8, 0))],
        out_specs=pl.BlockSpec((1, emb), lambda i: (0, 0)),
    )(table)
    return out.reshape((emb,))


# speedup vs baseline: 1.9181x; 1.0052x over previous
import jax
import jax.numpy as jnp
from jax.experimental import pallas as pl
from jax.experimental.pallas import tpu as pltpu


def kernel(x, table):
    row = x.shape[1]
    emb = table.shape[1]

    def body(table_hbm, out_ref, sem):
        pltpu.make_async_copy(
            table_hbm.at[pl.ds(row, 1), :], out_ref, sem
        ).start()
        pltpu.make_async_copy(
            table_hbm.at[pl.ds(row, 1), :], out_ref, sem
        ).wait()

    out = pl.pallas_call(
        body,
        out_shape=jax.ShapeDtypeStruct((1, emb), table.dtype),
        in_specs=[pl.BlockSpec(memory_space=pl.ANY)],
        scratch_shapes=[pltpu.SemaphoreType.DMA],
    )(table)
    return out.reshape((emb,))
